# baseline (speedup 1.0000x reference)
import functools

import jax
import jax.numpy as jnp
from jax import lax
from jax.experimental import pallas as pl
from jax.experimental.pallas import tpu as pltpu
from jax.experimental.pallas import tpu_sc as plsc

L = 16
GR = 8


@functools.lru_cache(maxsize=None)
def _build(U, I, K, B):
    NC, NS = 2, 16
    NW = NC * NS
    assert B % (NW * L) == 0
    PB = B // NW
    PCH = PB // L
    assert I % NS == 0
    WROWS = I // NS
    n_full = K // L
    rem = K % L
    offs = [i * L for i in range(n_full)]
    if rem:
        offs.append(K - L)
    NOFF = len(offs)
    KP = NOFF * L
    IDX_MINOR = 128
    assert (PB * KP) % IDX_MINOR == 0
    IDX_MAJOR = PB * KP // IDX_MINOR
    assert PB % GR == 0
    NG = PB // GR

    mesh = plsc.VectorSubcoreMesh(core_axis_name="c", subcore_axis_name="s")

    @functools.partial(
        pl.kernel,
        out_type=jax.ShapeDtypeStruct((B,), jnp.float32),
        mesh=mesh,
        compiler_params=pltpu.CompilerParams(needs_layout_passes=False),
        scratch_types=[
            pltpu.VMEM((PB + L,), jnp.int32),
            pltpu.VMEM((PB + L,), jnp.int32),
            pltpu.VMEM((PB,), jnp.int32),
            pltpu.VMEM((PB + L,), jnp.float32),
            pltpu.VMEM((PB,), jnp.float32),
            pltpu.VMEM((I,), jnp.float32),
            pltpu.VMEM((PB, K), jnp.int32),
            pltpu.VMEM((GR, I), jnp.float32),
            pltpu.VMEM_SHARED((I * I,), jnp.float32),
            pltpu.VMEM((IDX_MAJOR, IDX_MINOR), jnp.int32),
            pltpu.VMEM((IDX_MAJOR, IDX_MINOR), jnp.float32),
            pltpu.VMEM((IDX_MAJOR, IDX_MINOR), jnp.float32),
            pltpu.VMEM((K, PB), jnp.float32),
            pltpu.VMEM((PB,), jnp.float32),
            pltpu.SemaphoreType.DMA,
            pltpu.SemaphoreType.DMA,
            pltpu.SemaphoreType.DMA,
        ],
    )
    def launch(uidx_hbm, tidx_hbm, q2d_hbm, rm_hbm, w_hbm, buser_hbm,
               bitem_hbm, out_hbm,
               u_v, t_v, base_v, bu_v, bi_v, bitem_v, q_v, rrow_v, w_sh,
               widx_v, radj_v, w_v, t_prod, out_v, sem0, sem1, sem2):
        cid = lax.axis_index("c")
        sid = lax.axis_index("s")
        wid = sid * NC + cid
        base = wid * PB

        pltpu.sync_copy(uidx_hbm.at[pl.ds(base, PB)], u_v.at[pl.ds(0, PB)])
        pltpu.sync_copy(tidx_hbm.at[pl.ds(base, PB)], t_v.at[pl.ds(0, PB)])
        pltpu.sync_copy(bitem_hbm, bitem_v)
        pltpu.sync_copy(buser_hbm, rrow_v.at[0, pl.ds(0, U)])

        lanes = lax.iota(jnp.int32, L)
        zeros = lanes * 0

        def p1(i, _):
            sl = pl.ds(i * L, L)
            uvec = u_v[sl]
            tvec = t_v[sl]
            base_v[sl] = uvec * I + tvec
            bu_v[sl] = plsc.load_gather(rrow_v, [zeros, uvec])
            bi_v[sl] = plsc.load_gather(bitem_v, [tvec])
            return 0

        lax.fori_loop(0, PCH, p1, 0)

        qcopies = []
        for c in range(PCH):
            rvec = base_v[pl.ds(c * L, L)]
            for lane in range(L):
                j = c * L + lane
                qcopies.append(
                    pltpu.async_copy(q2d_hbm.at[rvec[lane]], q_v.at[j], sem0))

        wcopies = []
        for rr in range(WROWS):
            r = sid * WROWS + rr
            wcopies.append(pltpu.async_copy(
                w_hbm.at[r], w_sh.at[pl.ds(r * I, I)], sem2))

        for cp in qcopies:
            cp.wait()

        def p3(g, _):
            gsl = pl.ds(g * GR, L)
            uchunk = u_v[gsl]
            tchunk = t_v[gsl]
            buchunk = bu_v[gsl]
            rcopies = []
            for s in range(GR):
                rcopies.append(
                    pltpu.async_copy(rm_hbm.at[uchunk[s]], rrow_v.at[s], sem1))
            for cp in rcopies:
                cp.wait()
            for s in range(GR):
                j = g * GR + s
                t_j = tchunk[s]
                bu_j = buchunk[s]
                svec = zeros + s
                for ci, off in enumerate(offs):
                    qv = q_v[j, pl.ds(off, L)]
                    p = j * KP + ci * L
                    maj = p // IDX_MINOR
                    sl = pl.ds(p % IDX_MINOR, L)
                    widx_v[maj, sl] = qv * I + t_j
                    rv = plsc.load_gather(rrow_v, [svec, qv])
                    bj = plsc.load_gather(bitem_v, [qv])
                    radj_v[maj, sl] = rv - bu_j - bj
            return 0

        lax.fori_loop(0, NG, p3, 0)

        for cp in wcopies:
            cp.wait()
        plsc.subcore_barrier()
        gcopies = []
        for m in range(IDX_MAJOR):
            gcopies.append(pltpu.async_copy(
                w_sh.at[widx_v.at[m]], w_v.at[m], sem2))
        for cp in gcopies:
            cp.wait()

        def p5(c, _):
            p = c * L
            maj = p // IDX_MINOR
            sl = pl.ds(p % IDX_MINOR, L)
            prod = w_v[maj, sl] * radj_v[maj, sl]
            j = c // NOFF
            ci = c - j * NOFF
            off = jnp.minimum(ci * L, K - L)
            kv = off + lanes
            jv = zeros + j
            plsc.store_scatter(t_prod, [kv, jv], prod)
            return 0

        lax.fori_loop(0, PB * NOFF, p5, 0)

        def p6(o, _):
            sl = pl.ds(o * L, L)
            acc = t_prod[0, sl]
            for e in range(1, K):
                acc = acc + t_prod[e, sl]
            out_v[sl] = bu_v[sl] + bi_v[sl] + acc * (1.0 / K)
            return 0

        lax.fori_loop(0, PCH, p6, 0)
        pltpu.sync_copy(out_v, out_hbm.at[pl.ds(base, PB)])

    return launch


def kernel(user_idx, item_idx, qtus, rating_matrix, weight, b_user, b_item):
    U, I = rating_matrix.shape
    K = qtus.shape[-1]
    B = user_idx.shape[0]
    launch = _build(U, I, K, B)
    return launch(
        user_idx.astype(jnp.int32),
        item_idx.astype(jnp.int32),
        qtus.reshape(U * I, K),
        rating_matrix,
        weight,
        b_user,
        b_item,
    )


# speedup vs baseline: 4.6937x; 4.6937x over previous
import functools

import jax
import jax.numpy as jnp
from jax import lax
from jax.experimental import pallas as pl
from jax.experimental.pallas import tpu as pltpu
from jax.experimental.pallas import tpu_sc as plsc

L = 16


def _native_1d(x2d):
    R, C = x2d.shape
    z = x2d.reshape(R // 8, 8, C // 128, 128).transpose(0, 2, 1, 3)
    return z.reshape(R * C)


@functools.lru_cache(maxsize=None)
def _build(U, I, K, B):
    NC, NS = 2, 16
    NW = NC * NS
    assert B % (NW * L) == 0 and U % 8 == 0 and I % 128 == 0
    PB = B // NW
    PCH = PB // L
    NE = PB * K
    ECH = NE // L
    IDX_MINOR = 128
    assert NE % IDX_MINOR == 0
    IDX_MAJOR = NE // IDX_MINOR
    ROWS8 = 8 * I

    mesh = plsc.VectorSubcoreMesh(core_axis_name="c", subcore_axis_name="s")

    @functools.partial(
        pl.kernel,
        out_type=jax.ShapeDtypeStruct((B,), jnp.float32),
        mesh=mesh,
        compiler_params=pltpu.CompilerParams(needs_layout_passes=False),
        scratch_types=[
            pltpu.VMEM((PB,), jnp.int32),
            pltpu.VMEM((PB,), jnp.int32),
            pltpu.VMEM((PB,), jnp.int32),
            pltpu.VMEM((PB,), jnp.int32),
            pltpu.VMEM((PB,), jnp.int32),
            pltpu.VMEM((PB,), jnp.float32),
            pltpu.VMEM((PB,), jnp.float32),
            pltpu.VMEM((I,), jnp.float32),
            pltpu.VMEM((U,), jnp.float32),
            pltpu.VMEM((IDX_MAJOR, IDX_MINOR), jnp.int32),
            pltpu.VMEM((IDX_MAJOR, IDX_MINOR), jnp.int32),
            pltpu.VMEM((IDX_MAJOR, IDX_MINOR), jnp.int32),
            pltpu.VMEM((IDX_MAJOR, IDX_MINOR), jnp.int32),
            pltpu.VMEM((IDX_MAJOR, IDX_MINOR), jnp.float32),
            pltpu.VMEM((IDX_MAJOR, IDX_MINOR), jnp.float32),
            pltpu.VMEM((IDX_MAJOR, IDX_MINOR), jnp.float32),
            pltpu.VMEM((K, PB), jnp.float32),
            pltpu.VMEM((PB,), jnp.float32),
            pltpu.SemaphoreType.DMA,
            pltpu.SemaphoreType.DMA,
        ],
    )
    def launch(uidx_hbm, tidx_hbm, qn_hbm, rn_hbm, wn_hbm, buser_hbm,
               bitem_hbm, out_hbm,
               u_v, t_v, basep_v, upart_v, tpart_v, bu_v, bi_v, bitem_v,
               buser_v, qidx_v, q_v, widx_v, ridx_v, w_v, r_v, adj_v,
               t_prod, out_v, sem0, sem1):
        cid = lax.axis_index("c")
        sid = lax.axis_index("s")
        wid = sid * NC + cid
        base = wid * PB

        pltpu.sync_copy(uidx_hbm.at[pl.ds(base, PB)], u_v)
        pltpu.sync_copy(tidx_hbm.at[pl.ds(base, PB)], t_v)
        pltpu.sync_copy(buser_hbm, buser_v)
        pltpu.sync_copy(bitem_hbm, bitem_v)

        lanes = lax.iota(jnp.int32, L)

        def p1(i, _):
            sl = pl.ds(i * L, L)
            uvec = u_v[sl]
            tvec = t_v[sl]
            up = (uvec >> 3) * ROWS8 + (uvec & 7) * 128
            tp = (tvec >> 7) * 1024 + (tvec & 127)
            upart_v[sl] = up
            tpart_v[sl] = tp
            basep_v[sl] = up + tp
            bu_v[sl] = plsc.load_gather(buser_v, [uvec])
            bi_v[sl] = plsc.load_gather(bitem_v, [tvec])
            return 0

        lax.fori_loop(0, PCH, p1, 0)

        def p2(c, _):
            n_v = c * L + lanes
            jv = n_v // K
            kv = n_v - jv * K
            bp = plsc.load_gather(basep_v, [jv])
            p = c * L
            qidx_v[p // IDX_MINOR, pl.ds(p % IDX_MINOR, L)] = (
                kv * (U * I) + bp)
            return 0

        lax.fori_loop(0, ECH, p2, 0)
        qcopies = [
            pltpu.async_copy(qn_hbm.at[qidx_v.at[m]], q_v.at[m], sem0)
            for m in range(IDX_MAJOR)
        ]
        for cp in qcopies:
            cp.wait()

        def p3(c, _):
            p = c * L
            maj = p // IDX_MINOR
            sl = pl.ds(p % IDX_MINOR, L)
            n_v = p + lanes
            jv = n_v // K
            qv = q_v[maj, sl]
            tp = plsc.load_gather(tpart_v, [jv])
            up = plsc.load_gather(upart_v, [jv])
            bu_b = plsc.load_gather(bu_v, [jv])
            bj = plsc.load_gather(bitem_v, [qv])
            widx_v[maj, sl] = (qv >> 3) * ROWS8 + (qv & 7) * 128 + tp
            ridx_v[maj, sl] = up + (qv >> 7) * 1024 + (qv & 127)
            adj_v[maj, sl] = bu_b + bj
            return 0

        lax.fori_loop(0, ECH, p3, 0)

        copies = []
        for m in range(IDX_MAJOR):
            copies.append(pltpu.async_copy(
                wn_hbm.at[widx_v.at[m]], w_v.at[m], sem0))
            copies.append(pltpu.async_copy(
                rn_hbm.at[ridx_v.at[m]], r_v.at[m], sem1))
        for cp in copies:
            cp.wait()

        def p5(c, _):
            p = c * L
            maj = p // IDX_MINOR
            sl = pl.ds(p % IDX_MINOR, L)
            prod = w_v[maj, sl] * (r_v[maj, sl] - adj_v[maj, sl])
            n_v = p + lanes
            jv = n_v // K
            kv = n_v - jv * K
            plsc.store_scatter(t_prod, [kv, jv], prod)
            return 0

        lax.fori_loop(0, ECH, p5, 0)

        def p6(o, _):
            sl = pl.ds(o * L, L)
            acc = t_prod[0, sl]
            for e in range(1, K):
                acc = acc + t_prod[e, sl]
            out_v[sl] = bu_v[sl] + bi_v[sl] + acc * (1.0 / K)
            return 0

        lax.fori_loop(0, PCH, p6, 0)
        pltpu.sync_copy(out_v, out_hbm.at[pl.ds(base, PB)])

    return launch


def kernel(user_idx, item_idx, qtus, rating_matrix, weight, b_user, b_item):
    U, I = rating_matrix.shape
    K = qtus.shape[-1]
    B = user_idx.shape[0]
    launch = _build(U, I, K, B)
    q_native = _native_1d(qtus.transpose(2, 0, 1).reshape(K * U, I))
    return launch(
        user_idx.astype(jnp.int32),
        item_idx.astype(jnp.int32),
        q_native,
        _native_1d(rating_matrix),
        _native_1d(weight),
        b_user,
        b_item,
    )


# speedup vs baseline: 5.4388x; 1.1587x over previous
import functools

import jax
import jax.numpy as jnp
from jax import lax
from jax.experimental import pallas as pl
from jax.experimental.pallas import tpu as pltpu
from jax.experimental.pallas import tpu_sc as plsc

L = 16


def _native_1d(x2d):
    R, C = x2d.shape
    z = x2d.reshape(R // 8, 8, C // 128, 128).transpose(0, 2, 1, 3)
    return z.reshape(R * C)


@functools.lru_cache(maxsize=None)
def _build(U, I, K, B):
    NC, NS = 2, 16
    NW = NC * NS
    assert B % (NW * L) == 0 and U % 8 == 0 and I % 128 == 0
    PB = B // NW
    PCH = PB // L
    NE = PB * K
    ECH = NE // L
    IDX_MINOR = 128
    CPR = IDX_MINOR // L
    assert NE % IDX_MINOR == 0
    IDX_MAJOR = NE // IDX_MINOR
    HALF = (IDX_MAJOR + 1) // 2
    ROWS8 = 8 * I

    mesh = plsc.VectorSubcoreMesh(core_axis_name="c", subcore_axis_name="s")

    @functools.partial(
        pl.kernel,
        out_type=jax.ShapeDtypeStruct((B,), jnp.float32),
        mesh=mesh,
        compiler_params=pltpu.CompilerParams(needs_layout_passes=False),
        scratch_types=[
            pltpu.VMEM((PB,), jnp.int32),
            pltpu.VMEM((PB,), jnp.int32),
            pltpu.VMEM((PB,), jnp.int32),
            pltpu.VMEM((PB,), jnp.int32),
            pltpu.VMEM((PB,), jnp.int32),
            pltpu.VMEM((PB,), jnp.float32),
            pltpu.VMEM((PB,), jnp.float32),
            pltpu.VMEM((I,), jnp.float32),
            pltpu.VMEM((U,), jnp.float32),
            pltpu.VMEM((IDX_MAJOR, IDX_MINOR), jnp.int32),
            pltpu.VMEM((IDX_MAJOR, IDX_MINOR), jnp.int32),
            pltpu.VMEM((IDX_MAJOR, IDX_MINOR), jnp.int32),
            pltpu.VMEM((IDX_MAJOR, IDX_MINOR), jnp.int32),
            pltpu.VMEM((IDX_MAJOR, IDX_MINOR), jnp.int32),
            pltpu.VMEM((IDX_MAJOR, IDX_MINOR), jnp.int32),
            pltpu.VMEM((IDX_MAJOR, IDX_MINOR), jnp.float32),
            pltpu.VMEM((IDX_MAJOR, IDX_MINOR), jnp.float32),
            pltpu.VMEM((IDX_MAJOR, IDX_MINOR), jnp.float32),
            pltpu.VMEM((K, PB), jnp.float32),
            pltpu.VMEM((PB,), jnp.float32),
            pltpu.SemaphoreType.DMA,
            pltpu.SemaphoreType.DMA,
            pltpu.SemaphoreType.DMA,
            pltpu.SemaphoreType.DMA,
            pltpu.SemaphoreType.DMA,
            pltpu.SemaphoreType.DMA,
        ],
    )
    def launch(uidx_hbm, tidx_hbm, qn_hbm, rn_hbm, wn_hbm, buser_hbm,
               bitem_hbm, out_hbm,
               u_v, t_v, basep_v, upart_v, tpart_v, bu_v, bi_v, bitem_v,
               buser_v, jv_v, kv_v, qidx_v, q_v, widx_v, ridx_v, w_v, r_v,
               adj_v, t_prod, out_v, semq0, semq1, semw0, semw1, semr0,
               semr1):
        cid = lax.axis_index("c")
        sid = lax.axis_index("s")
        wid = sid * NC + cid
        base = wid * PB

        pltpu.sync_copy(uidx_hbm.at[pl.ds(base, PB)], u_v)
        pltpu.sync_copy(tidx_hbm.at[pl.ds(base, PB)], t_v)
        pltpu.sync_copy(buser_hbm, buser_v)
        pltpu.sync_copy(bitem_hbm, bitem_v)

        lanes = lax.iota(jnp.int32, L)
        halves = [(0, HALF, semq0, semw0, semr0),
                  (HALF, IDX_MAJOR, semq1, semw1, semr1)]

        @plsc.parallel_loop(0, ECH, unroll=4)
        def p0(c):
            n_v = c * L + lanes
            jv = n_v // K
            maj = c // CPR
            sl = pl.ds((c - maj * CPR) * L, L)
            jv_v[maj, sl] = jv
            kv_v[maj, sl] = n_v - jv * K

        @plsc.parallel_loop(0, PCH, unroll=2)
        def p1(i):
            sl = pl.ds(i * L, L)
            uvec = u_v[sl]
            tvec = t_v[sl]
            up = (uvec >> 3) * ROWS8 + (uvec & 7) * 128
            tp = (tvec >> 7) * 1024 + (tvec & 127)
            upart_v[sl] = up
            tpart_v[sl] = tp
            basep_v[sl] = up + tp
            bu_v[sl] = plsc.load_gather(buser_v, [uvec])
            bi_v[sl] = plsc.load_gather(bitem_v, [tvec])

        for m0, m1, semq, _, _ in halves:
            @plsc.parallel_loop(m0, m1)
            def p2(m, _semq=semq):
                for cc in range(CPR):
                    sl = pl.ds(cc * L, L)
                    bp = plsc.load_gather(basep_v, [jv_v[m, sl]])
                    qidx_v[m, sl] = kv_v[m, sl] * (U * I) + bp
                pltpu.async_copy(qn_hbm.at[qidx_v.at[m]], q_v.at[m], _semq)

        qdummy = qn_hbm.at[pl.ds(0, IDX_MINOR)]
        fdummy = wn_hbm.at[pl.ds(0, IDX_MINOR)]

        for m0, m1, semq, semw, semr in halves:
            for m in range(m0, m1):
                pltpu.make_async_copy(qdummy, q_v.at[m], semq).wait()

            @plsc.parallel_loop(m0, m1)
            def p3(m, _semw=semw, _semr=semr):
                for cc in range(CPR):
                    sl = pl.ds(cc * L, L)
                    jv = jv_v[m, sl]
                    qv = q_v[m, sl]
                    tp = plsc.load_gather(tpart_v, [jv])
                    up = plsc.load_gather(upart_v, [jv])
                    bu_b = plsc.load_gather(bu_v, [jv])
                    bj = plsc.load_gather(bitem_v, [qv])
                    widx_v[m, sl] = (qv >> 3) * ROWS8 + (qv & 7) * 128 + tp
                    ridx_v[m, sl] = up + (qv >> 7) * 1024 + (qv & 127)
                    adj_v[m, sl] = bu_b + bj
                pltpu.async_copy(wn_hbm.at[widx_v.at[m]], w_v.at[m], _semw)
                pltpu.async_copy(rn_hbm.at[ridx_v.at[m]], r_v.at[m], _semr)

        for m0, m1, _, semw, semr in halves:
            for m in range(m0, m1):
                pltpu.make_async_copy(fdummy, w_v.at[m], semw).wait()
                pltpu.make_async_copy(fdummy, r_v.at[m], semr).wait()

            @plsc.parallel_loop(m0 * CPR, m1 * CPR, unroll=2)
            def p5(c):
                maj = c // CPR
                sl = pl.ds((c - maj * CPR) * L, L)
                prod = w_v[maj, sl] * (r_v[maj, sl] - adj_v[maj, sl])
                plsc.store_scatter(
                    t_prod, [kv_v[maj, sl], jv_v[maj, sl]], prod)

        @plsc.parallel_loop(0, PCH)
        def p6(o):
            sl = pl.ds(o * L, L)
            acc = t_prod[0, sl]
            for e in range(1, K):
                acc = acc + t_prod[e, sl]
            out_v[sl] = bu_v[sl] + bi_v[sl] + acc * (1.0 / K)

        pltpu.sync_copy(out_v, out_hbm.at[pl.ds(base, PB)])

    return launch


def kernel(user_idx, item_idx, qtus, rating_matrix, weight, b_user, b_item):
    U, I = rating_matrix.shape
    K = qtus.shape[-1]
    B = user_idx.shape[0]
    launch = _build(U, I, K, B)
    q_native = _native_1d(qtus.transpose(2, 0, 1).reshape(K * U, I))
    return launch(
        user_idx.astype(jnp.int32),
        item_idx.astype(jnp.int32),
        q_native,
        _native_1d(rating_matrix),
        _native_1d(weight),
        b_user,
        b_item,
    )
